# Initial kernel scaffold; baseline (speedup 1.0000x reference)
import jax
import jax.numpy as jnp
from jax.experimental import pallas as pl


def kernel(x, edge_index, W_lin, b_lin, weight):
    raise NotImplementedError("write your pallas kernel here")


# speedup vs baseline: 8.5479x; 8.5479x over previous
import functools

import jax
import jax.numpy as jnp
from jax import lax
from jax.experimental import pallas as pl
from jax.experimental.pallas import tpu as pltpu
from jax.experimental.pallas import tpu_sc as plsc

N_NODES = 10000
N_EDGES = 320000
D_IN = 128
D_PAD = 144
D_OUT = 128

NC = 2
NS = 16
NW = NC * NS
EDGES_PER_WORKER = N_EDGES // NW
CHUNK = 80
N_CHUNKS = EDGES_PER_WORKER // CHUNK
N_ACC = 10240
ROWS_PER_TILE = N_ACC // NS

_mesh = plsc.VectorSubcoreMesh(
    core_axis_name="c", subcore_axis_name="s", num_cores=NC, num_subcores=NS
)


@functools.partial(
    pl.kernel,
    out_type=jax.ShapeDtypeStruct((NC * N_ACC, D_PAD), jnp.float32),
    mesh=_mesh,
    scratch_types=[
        pltpu.VMEM((2, CHUNK), jnp.int32),
        pltpu.VMEM((N_CHUNKS, CHUNK), jnp.int32),
        pltpu.VMEM((2, CHUNK, D_PAD), jnp.float32),
        pltpu.VMEM_SHARED((N_ACC, D_PAD), jnp.float32),
        pltpu.SemaphoreType.DMA,
        pltpu.SemaphoreType.DMA,
        pltpu.SemaphoreType.DMA,
    ],
    compiler_params=pltpu.CompilerParams(use_tc_tiling_on_sc=False),
)
def _sc_aggregate(xp_hbm, row_hbm, col_hbm, acc_hbm, ridx_v, cidx_v, rows_v,
                  acc_sh, sem_g, sem_s, sem_i):
    c = lax.axis_index("c")
    s = lax.axis_index("s")
    wid = s * NC + c

    pltpu.sync_copy(col_hbm.at[pl.ds(wid * N_CHUNKS, N_CHUNKS)], cidx_v)
    pltpu.async_copy(row_hbm.at[wid * N_CHUNKS], ridx_v.at[0], sem_i)

    def _zero_row(i, carry):
        for j in range(D_PAD // 16):
            rows_v[0, i, pl.ds(j * 16, 16)] = jnp.zeros((16,), jnp.float32)
        return carry

    lax.fori_loop(0, CHUNK, _zero_row, 0)
    row_base = pl.multiple_of(s * ROWS_PER_TILE, 8)
    for k in range(ROWS_PER_TILE // CHUNK):
        pltpu.sync_copy(rows_v.at[0],
                        acc_sh.at[pl.ds(row_base + k * CHUNK, CHUNK)])
    plsc.subcore_barrier()

    pltpu.async_copy(xp_hbm.at[cidx_v.at[0]], rows_v.at[0], sem_g)

    def _wait_scatter():
        pltpu.make_async_copy(
            rows_v.at[0], acc_sh.at[ridx_v.at[0]], sem_s).wait()

    def _edge_chunk(i, carry):
        p = lax.rem(i, 2)
        q = 1 - p

        @pl.when(i + 1 < N_CHUNKS)
        def _prefetch():
            @pl.when(i >= 1)
            def _():
                _wait_scatter()
            pltpu.async_copy(xp_hbm.at[cidx_v.at[i + 1]], rows_v.at[q], sem_g)
            pltpu.async_copy(row_hbm.at[wid * N_CHUNKS + i + 1], ridx_v.at[q],
                             sem_i)

        pltpu.make_async_copy(
            xp_hbm.at[cidx_v.at[i]], rows_v.at[p], sem_g).wait()
        pltpu.make_async_copy(
            row_hbm.at[wid * N_CHUNKS], ridx_v.at[p], sem_i).wait()
        pltpu.async_copy(rows_v.at[p], acc_sh.at[ridx_v.at[p]], sem_s,
                         add=True)
        return carry

    lax.fori_loop(0, N_CHUNKS, _edge_chunk, 0)
    _wait_scatter()
    _wait_scatter()
    plsc.subcore_barrier()

    out_base = pl.multiple_of(c * N_ACC + row_base, 8)
    pltpu.sync_copy(acc_sh.at[pl.ds(row_base, ROWS_PER_TILE)],
                    acc_hbm.at[pl.ds(out_base, ROWS_PER_TILE)])


def _tc_finalize(acc_ref, wlin_ref, blin_ref, wsq_ref, out_ref):
    a = acc_ref[:N_NODES, :] + acc_ref[N_ACC:N_ACC + N_NODES, :]
    feat = a[:, :D_IN]
    deg = jnp.maximum(a[:, D_IN:D_IN + 1], 1.0)
    norm = feat / deg
    h = lax.dot_general(norm, wlin_ref[...], (((1,), (1,)), ((), ())),
                        preferred_element_type=jnp.float32)
    h = h + blin_ref[...]
    out_ref[...] = jnp.dot(h, wsq_ref[...], preferred_element_type=jnp.float32)


def kernel(x, edge_index, W_lin, b_lin, weight):
    row = edge_index[0].astype(jnp.int32).reshape(N_EDGES // CHUNK, CHUNK)
    col = edge_index[1].astype(jnp.int32).reshape(N_EDGES // CHUNK, CHUNK)
    xp = jnp.concatenate(
        [x, jnp.ones((N_NODES, D_PAD - D_IN), x.dtype)], axis=1)
    acc = _sc_aggregate(xp, row, col)
    out = pl.pallas_call(
        _tc_finalize,
        out_shape=jax.ShapeDtypeStruct((N_NODES, D_OUT), jnp.float32),
    )(acc, W_lin, b_lin.reshape(1, D_OUT), weight)
    return out


# speedup vs baseline: 11.2589x; 1.3172x over previous
import functools

import jax
import jax.numpy as jnp
from jax import lax
from jax.experimental import pallas as pl
from jax.experimental.pallas import tpu as pltpu
from jax.experimental.pallas import tpu_sc as plsc

N_NODES = 10000
N_EDGES = 320000
D_IN = 128
D_DEG = 16
D_OUT = 128

NC = 2
NS = 16
NW = NC * NS
EDGES_PER_WORKER = N_EDGES // NW
CHUNK = 80
N_CHUNKS = EDGES_PER_WORKER // CHUNK
N_ACC = 10240
ROWS_PER_TILE = N_ACC // NS

_mesh = plsc.VectorSubcoreMesh(
    core_axis_name="c", subcore_axis_name="s", num_cores=NC, num_subcores=NS
)


@functools.partial(
    pl.kernel,
    out_type=(
        jax.ShapeDtypeStruct((NC * N_ACC, D_IN), jnp.float32),
        jax.ShapeDtypeStruct((NC * N_ACC, D_DEG), jnp.float32),
    ),
    mesh=_mesh,
    scratch_types=[
        pltpu.VMEM((2, CHUNK), jnp.int32),
        pltpu.VMEM((EDGES_PER_WORKER,), jnp.int32),
        pltpu.VMEM((2, CHUNK, D_IN), jnp.float32),
        pltpu.VMEM((CHUNK, D_DEG), jnp.float32),
        pltpu.VMEM((CHUNK, D_DEG), jnp.float32),
        pltpu.VMEM_SHARED((N_ACC, D_IN), jnp.float32),
        pltpu.VMEM_SHARED((N_ACC, D_DEG), jnp.float32),
        pltpu.SemaphoreType.DMA,
        pltpu.SemaphoreType.DMA,
        pltpu.SemaphoreType.DMA,
        pltpu.SemaphoreType.DMA,
    ],
    compiler_params=pltpu.CompilerParams(use_tc_tiling_on_sc=False),
)
def _sc_aggregate(x_hbm, edge_hbm, feat_hbm, deg_hbm, ridx_v, cidx_v, rows_v,
                  ones_v, zd_v, feat_sh, deg_sh, sem_g, sem_s, sem_d, sem_i):
    c = lax.axis_index("c")
    s = lax.axis_index("s")
    wid = s * NC + c
    ebase = wid * EDGES_PER_WORKER

    pltpu.sync_copy(edge_hbm.at[1, pl.ds(ebase, EDGES_PER_WORKER)], cidx_v)
    pltpu.async_copy(edge_hbm.at[0, pl.ds(ebase, CHUNK)], ridx_v.at[0], sem_i)

    def _zero_row(i, carry):
        for j in range(D_IN // 16):
            rows_v[0, i, pl.ds(j * 16, 16)] = jnp.zeros((16,), jnp.float32)
        ones_v[i, :] = jnp.ones((D_DEG,), jnp.float32)
        zd_v[i, :] = jnp.zeros((D_DEG,), jnp.float32)
        return carry

    lax.fori_loop(0, CHUNK, _zero_row, 0)
    row_base = pl.multiple_of(s * ROWS_PER_TILE, 8)
    for k in range(ROWS_PER_TILE // CHUNK):
        pltpu.sync_copy(rows_v.at[0],
                        feat_sh.at[pl.ds(row_base + k * CHUNK, CHUNK)])
        pltpu.sync_copy(zd_v, deg_sh.at[pl.ds(row_base + k * CHUNK, CHUNK)])
    plsc.subcore_barrier()

    pltpu.async_copy(x_hbm.at[cidx_v.at[pl.ds(0, CHUNK)]], rows_v.at[0], sem_g)

    def _wait_scatters():
        pltpu.make_async_copy(
            rows_v.at[0], feat_sh.at[ridx_v.at[0]], sem_s).wait()
        pltpu.make_async_copy(
            ones_v, deg_sh.at[ridx_v.at[0]], sem_d).wait()

    def _edge_chunk(i, carry):
        p = lax.rem(i, 2)
        q = 1 - p

        @pl.when(i + 1 < N_CHUNKS)
        def _prefetch():
            @pl.when(i >= 1)
            def _():
                _wait_scatters()
            off = pl.multiple_of((i + 1) * CHUNK, 8)
            pltpu.async_copy(x_hbm.at[cidx_v.at[pl.ds(off, CHUNK)]],
                             rows_v.at[q], sem_g)
            pltpu.async_copy(edge_hbm.at[0, pl.ds(ebase + off, CHUNK)],
                             ridx_v.at[q], sem_i)

        pltpu.make_async_copy(
            x_hbm.at[cidx_v.at[pl.ds(0, CHUNK)]], rows_v.at[p], sem_g).wait()
        pltpu.make_async_copy(
            edge_hbm.at[0, pl.ds(ebase, CHUNK)], ridx_v.at[p], sem_i).wait()
        pltpu.async_copy(rows_v.at[p], feat_sh.at[ridx_v.at[p]], sem_s,
                         add=True)
        pltpu.async_copy(ones_v, deg_sh.at[ridx_v.at[p]], sem_d, add=True)
        return carry

    lax.fori_loop(0, N_CHUNKS, _edge_chunk, 0)
    _wait_scatters()
    _wait_scatters()
    plsc.subcore_barrier()

    out_base = pl.multiple_of(c * N_ACC + row_base, 8)
    pltpu.sync_copy(feat_sh.at[pl.ds(row_base, ROWS_PER_TILE)],
                    feat_hbm.at[pl.ds(out_base, ROWS_PER_TILE)])
    pltpu.sync_copy(deg_sh.at[pl.ds(row_base, ROWS_PER_TILE)],
                    deg_hbm.at[pl.ds(out_base, ROWS_PER_TILE)])


def _tc_finalize(feat_ref, deg_ref, wlin_ref, blin_ref, wsq_ref, out_ref):
    f = feat_ref[:N_NODES, :] + feat_ref[N_ACC:N_ACC + N_NODES, :]
    d = deg_ref[:N_NODES, 0:1] + deg_ref[N_ACC:N_ACC + N_NODES, 0:1]
    norm = f / jnp.maximum(d, 1.0)
    h = lax.dot_general(norm, wlin_ref[...], (((1,), (1,)), ((), ())),
                        preferred_element_type=jnp.float32)
    h = h + blin_ref[...]
    out_ref[...] = jnp.dot(h, wsq_ref[...], preferred_element_type=jnp.float32)


def kernel(x, edge_index, W_lin, b_lin, weight):
    feat, deg = _sc_aggregate(x, edge_index.astype(jnp.int32))
    out = pl.pallas_call(
        _tc_finalize,
        out_shape=jax.ShapeDtypeStruct((N_NODES, D_OUT), jnp.float32),
    )(feat, deg, W_lin, b_lin.reshape(1, D_OUT), weight)
    return out


# speedup vs baseline: 13.0261x; 1.1570x over previous
import functools

import jax
import jax.numpy as jnp
from jax import lax
from jax.experimental import pallas as pl
from jax.experimental.pallas import tpu as pltpu
from jax.experimental.pallas import tpu_sc as plsc

N_NODES = 10000
N_EDGES = 320000
D_IN = 128
D_DEG = 16
D_OUT = 128

NC = 2
NS = 16
NW = NC * NS
EDGES_PER_WORKER = N_EDGES // NW
CHUNK = 80
N_CHUNKS = EDGES_PER_WORKER // CHUNK
N_ACC = 10240
ROWS_PER_TILE = N_ACC // NS
NROW = 3
NIDX = 4

_mesh = plsc.VectorSubcoreMesh(
    core_axis_name="c", subcore_axis_name="s", num_cores=NC, num_subcores=NS
)


@functools.partial(
    pl.kernel,
    out_type=(
        jax.ShapeDtypeStruct((NC * N_ACC, D_IN), jnp.float32),
        jax.ShapeDtypeStruct((NC * N_ACC, D_DEG), jnp.float32),
    ),
    mesh=_mesh,
    scratch_types=[
        pltpu.VMEM((NIDX, CHUNK), jnp.int32),
        pltpu.VMEM((NIDX, CHUNK), jnp.int32),
        pltpu.VMEM((NROW, CHUNK, D_IN), jnp.float32),
        pltpu.VMEM((CHUNK, D_DEG), jnp.float32),
        pltpu.VMEM((CHUNK, D_DEG), jnp.float32),
        pltpu.VMEM_SHARED((N_ACC, D_IN), jnp.float32),
        pltpu.VMEM_SHARED((N_ACC, D_DEG), jnp.float32),
        pltpu.SemaphoreType.DMA,
        pltpu.SemaphoreType.DMA,
        pltpu.SemaphoreType.DMA,
        pltpu.SemaphoreType.DMA,
        pltpu.SemaphoreType.DMA,
    ],
    compiler_params=pltpu.CompilerParams(use_tc_tiling_on_sc=False),
)
def _sc_aggregate(x_hbm, edge_hbm, feat_hbm, deg_hbm, ridx_v, cidx_v, rows_v,
                  ones_v, zd_v, feat_sh, deg_sh, sem_g, sem_s, sem_d, sem_ir,
                  sem_ic):
    c = lax.axis_index("c")
    s = lax.axis_index("s")
    wid = s * NC + c
    ebase = wid * EDGES_PER_WORKER

    def _load_idx(i, b):
        off = ebase + i * CHUNK
        pltpu.async_copy(edge_hbm.at[0, pl.ds(off, CHUNK)], ridx_v.at[b],
                         sem_ir)
        pltpu.async_copy(edge_hbm.at[1, pl.ds(off, CHUNK)], cidx_v.at[b],
                         sem_ic)

    def _wait_cidx():
        pltpu.make_async_copy(edge_hbm.at[1, pl.ds(0, CHUNK)], cidx_v.at[0],
                              sem_ic).wait()

    def _wait_ridx():
        pltpu.make_async_copy(edge_hbm.at[0, pl.ds(0, CHUNK)], ridx_v.at[0],
                              sem_ir).wait()

    def _gather(i, b):
        pltpu.async_copy(x_hbm.at[cidx_v.at[lax.rem(i, NIDX)]], rows_v.at[b],
                         sem_g)

    def _wait_gather():
        pltpu.make_async_copy(x_hbm.at[cidx_v.at[0]], rows_v.at[0],
                              sem_g).wait()

    def _wait_scatters():
        pltpu.make_async_copy(
            rows_v.at[0], feat_sh.at[ridx_v.at[0]], sem_s).wait()
        pltpu.make_async_copy(
            ones_v, deg_sh.at[ridx_v.at[0]], sem_d).wait()

    _load_idx(0, 0)
    _load_idx(1, 1)

    def _zero_row(i, carry):
        for j in range(D_IN // 16):
            rows_v[0, i, pl.ds(j * 16, 16)] = jnp.zeros((16,), jnp.float32)
        ones_v[i, :] = jnp.ones((D_DEG,), jnp.float32)
        zd_v[i, :] = jnp.zeros((D_DEG,), jnp.float32)
        return carry

    lax.fori_loop(0, CHUNK, _zero_row, 0)
    row_base = pl.multiple_of(s * ROWS_PER_TILE, 8)
    for k in range(ROWS_PER_TILE // CHUNK):
        pltpu.sync_copy(rows_v.at[0],
                        feat_sh.at[pl.ds(row_base + k * CHUNK, CHUNK)])
        pltpu.sync_copy(zd_v, deg_sh.at[pl.ds(row_base + k * CHUNK, CHUNK)])
    plsc.subcore_barrier()

    _wait_cidx()
    _gather(0, 0)

    def _edge_chunk(i, carry):
        @pl.when(i >= 2)
        def _():
            _wait_scatters()

        @pl.when(i + 1 < N_CHUNKS)
        def _():
            _wait_cidx()
            _gather(i + 1, lax.rem(i + 1, NROW))

        @pl.when(i + 2 < N_CHUNKS)
        def _():
            _load_idx(i + 2, lax.rem(i + 2, NIDX))

        _wait_gather()
        _wait_ridx()
        p = lax.rem(i, NROW)
        b = lax.rem(i, NIDX)
        pltpu.async_copy(rows_v.at[p], feat_sh.at[ridx_v.at[b]], sem_s,
                         add=True)
        pltpu.async_copy(ones_v, deg_sh.at[ridx_v.at[b]], sem_d, add=True)
        return carry

    lax.fori_loop(0, N_CHUNKS, _edge_chunk, 0)
    _wait_scatters()
    _wait_scatters()
    plsc.subcore_barrier()

    out_base = pl.multiple_of(c * N_ACC + row_base, 8)
    pltpu.sync_copy(feat_sh.at[pl.ds(row_base, ROWS_PER_TILE)],
                    feat_hbm.at[pl.ds(out_base, ROWS_PER_TILE)])
    pltpu.sync_copy(deg_sh.at[pl.ds(row_base, ROWS_PER_TILE)],
                    deg_hbm.at[pl.ds(out_base, ROWS_PER_TILE)])


def _tc_finalize(feat_ref, deg_ref, wlin_ref, blin_ref, wsq_ref, out_ref):
    f = feat_ref[:N_NODES, :] + feat_ref[N_ACC:N_ACC + N_NODES, :]
    d = deg_ref[:N_NODES, 0:1] + deg_ref[N_ACC:N_ACC + N_NODES, 0:1]
    norm = f / jnp.maximum(d, 1.0)
    h = lax.dot_general(norm, wlin_ref[...], (((1,), (1,)), ((), ())),
                        preferred_element_type=jnp.float32)
    h = h + blin_ref[...]
    out_ref[...] = jnp.dot(h, wsq_ref[...], preferred_element_type=jnp.float32)


def kernel(x, edge_index, W_lin, b_lin, weight):
    feat, deg = _sc_aggregate(x, edge_index.astype(jnp.int32))
    out = pl.pallas_call(
        _tc_finalize,
        out_shape=jax.ShapeDtypeStruct((N_NODES, D_OUT), jnp.float32),
    )(feat, deg, W_lin, b_lin.reshape(1, D_OUT), weight)
    return out
